# baseline (speedup 1.0000x reference)
import jax
import jax.numpy as jnp
from jax import lax
from jax.experimental import pallas as pl
from jax.experimental.pallas import tpu as pltpu

_NUM_ANCHORS = 3
_NUM_CLASSES = 80
_ATTRS = 5 + _NUM_CLASSES
_W = 52
_S = 2704
_STRIDE = 8.0
_ANCHOR_W = (10.0, 16.0, 33.0)
_ANCHOR_H = (13.0, 30.0, 23.0)
_NBUF = 4
_LOOKAHEAD = 3
_NTILE = 96


def _decode_kernel(in_hbm, out_hbm, ibuf, obuf, isem, osem):
    def in_copy(u):
        b = u // _NUM_ANCHORS
        a = u % _NUM_ANCHORS
        r = lax.rem(u, _NBUF)
        return pltpu.make_async_copy(
            in_hbm.at[b, a], ibuf.at[r], isem.at[r]
        )

    def out_copy(u):
        b = u // _NUM_ANCHORS
        a = u % _NUM_ANCHORS
        r = lax.rem(u, _NBUF)
        return pltpu.make_async_copy(
            obuf.at[r], out_hbm.at[b, pl.ds(a * _S, _S), :], osem.at[r]
        )

    col = lax.broadcasted_iota(jnp.int32, (1, _S), 1)
    gx = (col % _W).astype(jnp.float32)
    gy = (col // _W).astype(jnp.float32)

    for u0 in range(_LOOKAHEAD):
        in_copy(u0).start()

    def body(u, _):
        nxt = u + _LOOKAHEAD

        @pl.when(nxt < _NTILE)
        def _():
            in_copy(nxt).start()

        @pl.when(u >= _NBUF)
        def _():
            out_copy(u - _NBUF).wait()

        in_copy(u).wait()

        a = u % _NUM_ANCHORS
        r = lax.rem(u, _NBUF)
        aw = jnp.where(a == 0, _ANCHOR_W[0],
                       jnp.where(a == 1, _ANCHOR_W[1], _ANCHOR_W[2]))
        ah = jnp.where(a == 0, _ANCHOR_H[0],
                       jnp.where(a == 1, _ANCHOR_H[1], _ANCHOR_H[2]))

        blk = ibuf[r]
        sig = jax.nn.sigmoid(blk)
        bx = (sig[0:1] + gx) * _STRIDE
        by = (sig[1:2] + gy) * _STRIDE
        bw = jnp.exp(blk[2:3]) * aw
        bh = jnp.exp(blk[3:4]) * ah
        dec = jnp.concatenate([bx, by, bw, bh, sig[4:]], axis=0)
        obuf[r] = dec.T

        out_copy(u).start()
        return 0

    lax.fori_loop(0, _NTILE, body, 0)

    for u0 in range(_NTILE - _NBUF, _NTILE):
        out_copy(u0).wait()


def kernel(input):
    bs = input.shape[0]
    flat = input.reshape(bs, _NUM_ANCHORS, _ATTRS, _S)
    return pl.pallas_call(
        _decode_kernel,
        in_specs=[pl.BlockSpec(memory_space=pl.ANY)],
        out_specs=pl.BlockSpec(memory_space=pl.ANY),
        out_shape=jax.ShapeDtypeStruct((bs, _NUM_ANCHORS * _S, _ATTRS), jnp.float32),
        scratch_shapes=[
            pltpu.VMEM((_NBUF, _ATTRS, _S), jnp.float32),
            pltpu.VMEM((_NBUF, _S, _ATTRS), jnp.float32),
            pltpu.SemaphoreType.DMA((_NBUF,)),
            pltpu.SemaphoreType.DMA((_NBUF,)),
        ],
    )(flat)


# speedup vs baseline: 1.0007x; 1.0007x over previous
import jax
import jax.numpy as jnp
from jax import lax
from jax.experimental import pallas as pl
from jax.experimental.pallas import tpu as pltpu

_NUM_ANCHORS = 3
_NUM_CLASSES = 80
_ATTRS = 5 + _NUM_CLASSES
_W = 52
_S = 2704
_STRIDE = 8.0
_ANCHOR_W = (10.0, 16.0, 33.0)
_ANCHOR_H = (13.0, 30.0, 23.0)
_NBUF = 4
_LOOKAHEAD = 3
_NTILE = 96
_NGROUP = _NTILE // _NBUF


def _decode_kernel(in_hbm, out_hbm, ibuf, obuf, isem, osem):
    def in_copy(u, slot):
        b = u // _NUM_ANCHORS
        a = u % _NUM_ANCHORS
        return pltpu.make_async_copy(in_hbm.at[b, a], ibuf.at[slot], isem.at[slot])

    def out_copy(u, slot):
        b = u // _NUM_ANCHORS
        a = u % _NUM_ANCHORS
        return pltpu.make_async_copy(
            obuf.at[slot], out_hbm.at[b, pl.ds(a * _S, _S), :], osem.at[slot]
        )

    col = lax.broadcasted_iota(jnp.int32, (1, _S), 1)
    gx = (col % _W).astype(jnp.float32)
    gy = (col // _W).astype(jnp.float32)

    for u0 in range(_LOOKAHEAD):
        in_copy(u0, u0).start()

    def body(i, _):
        for j in range(_NBUF):
            u = i * _NBUF + j

            @pl.when(u + _LOOKAHEAD < _NTILE)
            def _():
                in_copy(u + _LOOKAHEAD, (j + _LOOKAHEAD) % _NBUF).start()

            @pl.when(u >= _NBUF)
            def _():
                out_copy(u - _NBUF, j).wait()

            in_copy(u, j).wait()

            a = u % _NUM_ANCHORS
            aw = jnp.where(a == 0, _ANCHOR_W[0],
                           jnp.where(a == 1, _ANCHOR_W[1], _ANCHOR_W[2]))
            ah = jnp.where(a == 0, _ANCHOR_H[0],
                           jnp.where(a == 1, _ANCHOR_H[1], _ANCHOR_H[2]))

            blk = ibuf[j]
            sig = jax.nn.sigmoid(blk)
            bx = (sig[0:1] + gx) * _STRIDE
            by = (sig[1:2] + gy) * _STRIDE
            bw = jnp.exp(blk[2:3]) * aw
            bh = jnp.exp(blk[3:4]) * ah
            dec = jnp.concatenate([bx, by, bw, bh, sig[4:]], axis=0)
            obuf[j] = dec.T

            out_copy(u, j).start()
        return 0

    lax.fori_loop(0, _NGROUP, body, 0)

    for u0 in range(_NTILE - _NBUF, _NTILE):
        out_copy(u0, u0 % _NBUF).wait()


def kernel(input):
    bs = input.shape[0]
    flat = input.reshape(bs, _NUM_ANCHORS, _ATTRS, _S)
    return pl.pallas_call(
        _decode_kernel,
        in_specs=[pl.BlockSpec(memory_space=pl.ANY)],
        out_specs=pl.BlockSpec(memory_space=pl.ANY),
        out_shape=jax.ShapeDtypeStruct((bs, _NUM_ANCHORS * _S, _ATTRS), jnp.float32),
        scratch_shapes=[
            pltpu.VMEM((_NBUF, _ATTRS, _S), jnp.float32),
            pltpu.VMEM((_NBUF, _S, _ATTRS), jnp.float32),
            pltpu.SemaphoreType.DMA((_NBUF,)),
            pltpu.SemaphoreType.DMA((_NBUF,)),
        ],
    )(flat)


# speedup vs baseline: 1.9426x; 1.9413x over previous
import jax
import jax.numpy as jnp
from jax.experimental import pallas as pl

_NUM_ANCHORS = 3
_NUM_CLASSES = 80
_ATTRS = 5 + _NUM_CLASSES
_H = 52
_W = 52
_S = _H * _W
_STRIDE = 8.0
_ANCHOR_W = (10.0, 16.0, 33.0)
_ANCHOR_H = (13.0, 30.0, 23.0)


def _sigmoid(x):
    return 0.5 * jnp.tanh(0.5 * x) + 0.5


def _decode_kernel(in_ref, out_ref):
    col = jax.lax.broadcasted_iota(jnp.int32, (1, _S), 1)
    gx = (col % _W).astype(jnp.float32)
    gy = (col // _W).astype(jnp.float32)

    for i in range(2):
        for a in range(_NUM_ANCHORS):
            blk = in_ref[i, a * _ATTRS:(a + 1) * _ATTRS, :]
            sig = _sigmoid(blk)
            bx = (sig[0:1] + gx) * _STRIDE
            by = (sig[1:2] + gy) * _STRIDE
            bw = jnp.exp(blk[2:3]) * _ANCHOR_W[a]
            bh = jnp.exp(blk[3:4]) * _ANCHOR_H[a]
            dec = jnp.concatenate([bx, by, bw, bh, sig[4:]], axis=0)
            out_ref[i, a * _S:(a + 1) * _S, :] = dec.T


def kernel(input):
    bs = input.shape[0]
    flat = input.reshape(bs, _NUM_ANCHORS * _ATTRS, _S)
    out = pl.pallas_call(
        _decode_kernel,
        grid=(bs // 2,),
        in_specs=[pl.BlockSpec((2, _NUM_ANCHORS * _ATTRS, _S), lambda b: (b, 0, 0))],
        out_specs=pl.BlockSpec((2, _NUM_ANCHORS * _S, _ATTRS), lambda b: (b, 0, 0)),
        out_shape=jax.ShapeDtypeStruct((bs, _NUM_ANCHORS * _S, _ATTRS), jnp.float32),
    )(flat)
    return out


# speedup vs baseline: 1.9575x; 1.0077x over previous
import jax
import jax.numpy as jnp
from jax.experimental import pallas as pl

_NUM_ANCHORS = 3
_NUM_CLASSES = 80
_ATTRS = 5 + _NUM_CLASSES
_H = 52
_W = 52
_S = _H * _W
_STRIDE = 8.0
_ANCHOR_W = (10.0, 16.0, 33.0)
_ANCHOR_H = (13.0, 30.0, 23.0)


def _sigmoid(x):
    return 0.5 * jnp.tanh(0.5 * x) + 0.5


def _decode_kernel(in_ref, out_ref):
    col = jax.lax.broadcasted_iota(jnp.int32, (1, _S), 1)
    gx = (col % _W).astype(jnp.float32)
    gy = (col // _W).astype(jnp.float32)

    for i in range(4):
        for a in range(_NUM_ANCHORS):
            blk = in_ref[i, a * _ATTRS:(a + 1) * _ATTRS, :]
            sig = _sigmoid(blk)
            bx = (sig[0:1] + gx) * _STRIDE
            by = (sig[1:2] + gy) * _STRIDE
            bw = jnp.exp(blk[2:3]) * _ANCHOR_W[a]
            bh = jnp.exp(blk[3:4]) * _ANCHOR_H[a]
            dec = jnp.concatenate([bx, by, bw, bh, sig[4:]], axis=0)
            out_ref[i, a * _S:(a + 1) * _S, :] = dec.T


def kernel(input):
    bs = input.shape[0]
    flat = input.reshape(bs, _NUM_ANCHORS * _ATTRS, _S)
    out = pl.pallas_call(
        _decode_kernel,
        grid=(bs // 4,),
        in_specs=[pl.BlockSpec((4, _NUM_ANCHORS * _ATTRS, _S), lambda b: (b, 0, 0))],
        out_specs=pl.BlockSpec((4, _NUM_ANCHORS * _S, _ATTRS), lambda b: (b, 0, 0)),
        out_shape=jax.ShapeDtypeStruct((bs, _NUM_ANCHORS * _S, _ATTRS), jnp.float32),
    )(flat)
    return out
